# baseline (speedup 1.0000x reference)
import functools

import jax
import jax.numpy as jnp
from jax import lax
from jax.experimental import pallas as pl
from jax.experimental.pallas import tpu as pltpu
from jax.experimental.pallas import tpu_sc as plsc

_E = 8
_H = 1024
_K = 2
_B, _S = 4, 2048
_T = _B * _S
_A = _T * _K
_TM = 256
_P = _A + _E * _TM
_NT = _P // _TM
_RT = 1024

_NC, _NS = 2, 16
_NW = _NC * _NS
_TPW = _T // _NW
_DC = 32
_NDC = _TPW // _DC
_CT = 16

_f32 = jnp.float32
_i32 = jnp.int32


def _router_body(x_ref, w_ref, b_ref, e1_ref, e2_ref, w1_ref, w2_ref):
    s = jnp.dot(x_ref[...], w_ref[...], preferred_element_type=_f32,
                precision=lax.Precision.DEFAULT) + b_ref[...]
    m1 = jnp.max(s, axis=1, keepdims=True)
    den = jnp.sum(jnp.exp(s - m1), axis=1, keepdims=True)
    p = jnp.exp(s - m1) / den
    iot = lax.broadcasted_iota(_i32, s.shape, 1)
    p1 = jnp.max(p, axis=1, keepdims=True)
    id1 = jnp.min(jnp.where(p == p1, iot, _E), axis=1, keepdims=True)
    pm = jnp.where(iot == id1, -1.0, p)
    p2 = jnp.max(pm, axis=1, keepdims=True)
    id2 = jnp.min(jnp.where(pm == p2, iot, _E), axis=1, keepdims=True)
    tot = p1 + p2 + 1e-9
    e1_ref[...] = id1
    e2_ref[...] = id2
    w1_ref[...] = p1 / tot
    w2_ref[...] = p2 / tot


def _router(x2, router_W, router_b):
    return pl.pallas_call(
        _router_body,
        grid=(_T // _RT,),
        in_specs=[
            pl.BlockSpec((_RT, _H), lambda i: (i, 0)),
            pl.BlockSpec((_H, _E), lambda i: (0, 0)),
            pl.BlockSpec((1, _E), lambda i: (0, 0)),
        ],
        out_specs=[
            pl.BlockSpec((_RT, 1), lambda i: (i, 0)),
            pl.BlockSpec((_RT, 1), lambda i: (i, 0)),
            pl.BlockSpec((_RT, 1), lambda i: (i, 0)),
            pl.BlockSpec((_RT, 1), lambda i: (i, 0)),
        ],
        out_shape=[
            jax.ShapeDtypeStruct((_T, 1), _i32),
            jax.ShapeDtypeStruct((_T, 1), _i32),
            jax.ShapeDtypeStruct((_T, 1), _f32),
            jax.ShapeDtypeStruct((_T, 1), _f32),
        ],
    )(x2, router_W, router_b.reshape(1, _E))


_sc_mesh = plsc.VectorSubcoreMesh(core_axis_name="c", subcore_axis_name="s")


@functools.partial(
    pl.kernel,
    mesh=_sc_mesh,
    out_type=[
        jax.ShapeDtypeStruct((_P, _H // 2), _i32),
        jax.ShapeDtypeStruct((_P, 128), _f32),
    ],
    scratch_types=[
        pltpu.VMEM((_DC,), _i32), pltpu.VMEM((_DC,), _i32),
        pltpu.VMEM((_DC,), _i32), pltpu.VMEM((_DC,), _i32),
        pltpu.VMEM((_DC, _H), _i32), pltpu.VMEM((_DC, _H), _i32),
        pltpu.VMEM((_DC, _H // 2), _i32), pltpu.VMEM((_DC, _H // 2), _i32),
        pltpu.VMEM((_DC, 128), _f32), pltpu.VMEM((_DC, 128), _f32),
        pltpu.VMEM((_DC, 128), _f32), pltpu.VMEM((_DC, 128), _f32),
        pltpu.SemaphoreType.DMA, pltpu.SemaphoreType.DMA,
        pltpu.SemaphoreType.DMA, pltpu.SemaphoreType.DMA,
    ],
)
def _dispatch_scatter(x_hbm, p1_hbm, p2_hbm, w1_hbm, w2_hbm, xs_hbm, wp_hbm,
                      i1x, i1y, i2x, i2y, rx, ry, px, py, w1x, w1y, w2x, w2y,
                      sr0, sr1, ss0, ss1):
    wid = lax.axis_index("s") * _NC + lax.axis_index("c")
    base = wid * _TPW
    i1_v = (i1x, i1y)
    i2_v = (i2x, i2y)
    rows_v = (rx, ry)
    pk_v = (px, py)
    w1_v = (w1x, w1y)
    w2_v = (w2x, w2y)
    sr = (sr0, sr1)
    ss = (ss0, ss1)
    rh = [None] * _NDC
    sh = [None] * _NDC
    half = _H // 2

    def pack_chunk(b):
        def rne(u):
            return lax.shift_right_logical(
                u + jnp.int32(0x7FFF)
                + jnp.bitwise_and(lax.shift_right_logical(u, 16), jnp.int32(1)),
                16)

        def prow(t, carry):
            for j in range(0, half, 16):
                lo = rne(rows_v[b][t, pl.ds(j, 16)])
                hi = rne(rows_v[b][t, pl.ds(half + j, 16)])
                pk_v[b][t, pl.ds(j, 16)] = jnp.bitwise_or(
                    lo, lax.shift_left(hi, 16))
            return carry

        lax.fori_loop(0, _DC, prow, 0)

    def scatters(g):
        b = g & 1
        return [
            pltpu.async_copy(pk_v[b], xs_hbm.at[i1_v[b]], ss[b]),
            pltpu.async_copy(pk_v[b], xs_hbm.at[i2_v[b]], ss[b]),
            pltpu.async_copy(w1_v[b], wp_hbm.at[i1_v[b]], ss[b]),
            pltpu.async_copy(w2_v[b], wp_hbm.at[i2_v[b]], ss[b]),
        ]

    for g in range(_NDC):
        b = g & 1
        off = base + g * _DC
        if g >= 2:
            for h in sh[g - 2]:
                h.wait()
        pltpu.sync_copy(p1_hbm.at[pl.ds(off, _DC)], i1_v[b])
        pltpu.sync_copy(p2_hbm.at[pl.ds(off, _DC)], i2_v[b])
        pltpu.sync_copy(w1_hbm.at[pl.ds(off, _DC)], w1_v[b])
        pltpu.sync_copy(w2_hbm.at[pl.ds(off, _DC)], w2_v[b])
        rh[g] = pltpu.async_copy(x_hbm.at[pl.ds(off, _DC)], rows_v[b], sr[b])
        if g >= 1:
            rh[g - 1].wait()
            pack_chunk(1 - b)
            sh[g - 1] = scatters(g - 1)
    rh[_NDC - 1].wait()
    pack_chunk((_NDC - 1) & 1)
    sh[_NDC - 1] = scatters(_NDC - 1)
    for h in sh[_NDC - 2]:
        h.wait()
    for h in sh[_NDC - 1]:
        h.wait()


def _gemm_body(te_ref, valid_ref, xg_ref, w_ref, b_ref, wp_ref, out_ref):
    i = pl.program_id(0)

    @pl.when(valid_ref[i] != 0)
    def _():
        v = xg_ref[...]
        xlo = lax.bitcast_convert_type(lax.shift_left(v, 16), _f32)
        xhi = lax.bitcast_convert_type(
            jnp.bitwise_and(v, jnp.int32(-65536)), _f32)
        acc = (jnp.dot(xlo, w_ref[0, 0], preferred_element_type=_f32)
               + jnp.dot(xhi, w_ref[0, 1], preferred_element_type=_f32))
        out_ref[...] = (acc + b_ref[0]) * wp_ref[...][:, 0:1]


def _gemm(tile_eid, tile_valid, xs, expert_W, expert_b, wp):
    grid_spec = pltpu.PrefetchScalarGridSpec(
        num_scalar_prefetch=2,
        grid=(_NT,),
        in_specs=[
            pl.BlockSpec((_TM, _H // 2), lambda i, te, va: (i, 0)),
            pl.BlockSpec((1, 2, _H // 2, _H), lambda i, te, va: (te[i], 0, 0, 0)),
            pl.BlockSpec((1, 1, _H), lambda i, te, va: (te[i], 0, 0)),
            pl.BlockSpec((_TM, 128), lambda i, te, va: (i, 0)),
        ],
        out_specs=pl.BlockSpec((_TM, _H), lambda i, te, va: (i, 0)),
    )
    return pl.pallas_call(
        _gemm_body,
        grid_spec=grid_spec,
        out_shape=jax.ShapeDtypeStruct((_P, _H), _f32),
    )(tile_eid, tile_valid, xs, expert_W.reshape(_E, 2, _H // 2, _H),
      expert_b.reshape(_E, 1, _H), wp)


@functools.partial(
    pl.kernel,
    mesh=_sc_mesh,
    out_type=jax.ShapeDtypeStruct((_T, _H), _f32),
    scratch_types=[
        pltpu.VMEM((_CT,), _i32), pltpu.VMEM((_CT,), _i32),
        pltpu.VMEM((_CT,), _i32), pltpu.VMEM((_CT,), _i32),
        pltpu.VMEM((_CT, _H), _f32), pltpu.VMEM((_CT, _H), _f32),
        pltpu.VMEM((_CT, _H), _f32), pltpu.VMEM((_CT, _H), _f32),
        pltpu.SemaphoreType.DMA, pltpu.SemaphoreType.DMA,
        pltpu.SemaphoreType.DMA, pltpu.SemaphoreType.DMA,
    ],
)
def _combine(yp_hbm, p1_hbm, p2_hbm, out_hbm,
             i1x, i1y, i2x, i2y, ax, ay, bx, by, sg0, sg1, so0, so1):
    wid = lax.axis_index("s") * _NC + lax.axis_index("c")
    base = wid * _TPW
    i1_v = (i1x, i1y)
    i2_v = (i2x, i2y)
    a_v = (ax, ay)
    b_v = (bx, by)
    sg = (sg0, sg1)
    so = (so0, so1)
    nch = _TPW // _CT

    def compute(b):
        def trow(t, carry):
            for c in range(0, _H, 16):
                sl = pl.ds(c, 16)
                a_v[b][t, sl] = a_v[b][t, sl] + b_v[b][t, sl]
            return carry

        lax.fori_loop(0, _CT, trow, 0)

    gha = [None] * nch
    ghb = [None] * nch
    wh = [None] * nch
    for g in range(nch):
        b = g & 1
        off = base + g * _CT
        if g >= 2:
            wh[g - 2].wait()
        pltpu.sync_copy(p1_hbm.at[pl.ds(off, _CT)], i1_v[b])
        pltpu.sync_copy(p2_hbm.at[pl.ds(off, _CT)], i2_v[b])
        gha[g] = pltpu.async_copy(yp_hbm.at[i1_v[b]], a_v[b], sg[b])
        ghb[g] = pltpu.async_copy(yp_hbm.at[i2_v[b]], b_v[b], sg[b])
        if g >= 1:
            gha[g - 1].wait()
            ghb[g - 1].wait()
            compute(1 - b)
            wh[g - 1] = pltpu.async_copy(
                a_v[1 - b], out_hbm.at[pl.ds(off - _CT, _CT)], so[1 - b])
    bl = (nch - 1) & 1
    gha[nch - 1].wait()
    ghb[nch - 1].wait()
    compute(bl)
    wh[nch - 1] = pltpu.async_copy(
        a_v[bl], out_hbm.at[pl.ds(base + (nch - 1) * _CT, _CT)], so[bl])
    wh[nch - 2].wait()
    wh[nch - 1].wait()


def kernel(x, router_W, router_b, expert_W, expert_b):
    x2 = x.reshape(_T, _H)
    e1o, e2o, w1o, w2o = _router(x2, router_W, router_b)
    e1 = e1o.reshape(_T)
    e2 = e2o.reshape(_T)

    ea = jnp.concatenate([e1, e2])
    oh = (ea[:, None] == jnp.arange(_E, dtype=_i32)[None, :]).astype(_i32)
    ranks = jnp.cumsum(oh, axis=0)
    counts = ranks[-1]
    padded = ((counts + _TM - 1) // _TM) * _TM
    pend = jnp.cumsum(padded)
    poff = pend - padded
    rank_a = jnp.take_along_axis(ranks, ea[:, None], axis=1)[:, 0]
    pos = poff[ea] + rank_a - 1
    tile_start = jnp.arange(_NT, dtype=_i32) * _TM
    tile_e = jnp.sum((tile_start[:, None] >= pend[None, :]).astype(_i32), axis=1)
    tile_eid = jnp.minimum(tile_e, _E - 1).astype(_i32)
    tile_valid = (tile_start < pend[-1]).astype(_i32)
    pos1 = pos[:_T]
    pos2 = pos[_T:]
    w1w = jnp.broadcast_to(w1o, (_T, 128))
    w2w = jnp.broadcast_to(w2o, (_T, 128))

    x_i = lax.bitcast_convert_type(x2, _i32)
    xs, wp = _dispatch_scatter(x_i, pos1, pos2, w1w, w2w)
    yp = _gemm(tile_eid, tile_valid, xs, expert_W, expert_b, wp)
    out2 = _combine(yp, pos1, pos2)
    return out2.reshape(_B, _S, _H)


# speedup vs baseline: 1.1190x; 1.1190x over previous
import functools

import jax
import jax.numpy as jnp
from jax import lax
from jax.experimental import pallas as pl
from jax.experimental.pallas import tpu as pltpu
from jax.experimental.pallas import tpu_sc as plsc

_E = 8
_H = 1024
_K = 2
_B, _S = 4, 2048
_T = _B * _S
_A = _T * _K
_TM = 256
_P = _A + _E * _TM
_NT = _P // _TM
_RT = 1024

_NC, _NS = 2, 16
_NW = _NC * _NS
_TPW = _T // _NW
_DC = 32
_NDC = _TPW // _DC
_CT = 16

_f32 = jnp.float32
_i32 = jnp.int32


def _router_body(x_ref, w_ref, b_ref, e1_ref, e2_ref, w1_ref, w2_ref):
    s = jnp.dot(x_ref[...], w_ref[...], preferred_element_type=_f32,
                precision=lax.Precision.DEFAULT) + b_ref[...]
    m1 = jnp.max(s, axis=1, keepdims=True)
    den = jnp.sum(jnp.exp(s - m1), axis=1, keepdims=True)
    p = jnp.exp(s - m1) / den
    iot = lax.broadcasted_iota(_i32, s.shape, 1)
    p1 = jnp.max(p, axis=1, keepdims=True)
    id1 = jnp.min(jnp.where(p == p1, iot, _E), axis=1, keepdims=True)
    pm = jnp.where(iot == id1, -1.0, p)
    p2 = jnp.max(pm, axis=1, keepdims=True)
    id2 = jnp.min(jnp.where(pm == p2, iot, _E), axis=1, keepdims=True)
    tot = p1 + p2 + 1e-9
    e1_ref[...] = id1
    e2_ref[...] = id2
    w1_ref[...] = p1 / tot
    w2_ref[...] = p2 / tot


def _router(x2, router_W, router_b):
    return pl.pallas_call(
        _router_body,
        grid=(_T // _RT,),
        in_specs=[
            pl.BlockSpec((_RT, _H), lambda i: (i, 0)),
            pl.BlockSpec((_H, _E), lambda i: (0, 0)),
            pl.BlockSpec((1, _E), lambda i: (0, 0)),
        ],
        out_specs=[
            pl.BlockSpec((_RT, 1), lambda i: (i, 0)),
            pl.BlockSpec((_RT, 1), lambda i: (i, 0)),
            pl.BlockSpec((_RT, 1), lambda i: (i, 0)),
            pl.BlockSpec((_RT, 1), lambda i: (i, 0)),
        ],
        out_shape=[
            jax.ShapeDtypeStruct((_T, 1), _i32),
            jax.ShapeDtypeStruct((_T, 1), _i32),
            jax.ShapeDtypeStruct((_T, 1), _f32),
            jax.ShapeDtypeStruct((_T, 1), _f32),
        ],
    )(x2, router_W, router_b.reshape(1, _E))


_sc_mesh = plsc.VectorSubcoreMesh(core_axis_name="c", subcore_axis_name="s")


@functools.partial(
    pl.kernel,
    mesh=_sc_mesh,
    out_type=[
        jax.ShapeDtypeStruct((_P, _H), _f32),
        jax.ShapeDtypeStruct((_P, 128), _f32),
    ],
    scratch_types=[
        pltpu.VMEM((_DC,), _i32), pltpu.VMEM((_DC,), _i32),
        pltpu.VMEM((_DC,), _i32), pltpu.VMEM((_DC,), _i32),
        pltpu.VMEM((_DC, _H), _f32), pltpu.VMEM((_DC, _H), _f32),
        pltpu.VMEM((_DC, 128), _f32), pltpu.VMEM((_DC, 128), _f32),
        pltpu.VMEM((_DC, 128), _f32), pltpu.VMEM((_DC, 128), _f32),
        pltpu.SemaphoreType.DMA, pltpu.SemaphoreType.DMA,
        pltpu.SemaphoreType.DMA, pltpu.SemaphoreType.DMA,
    ],
)
def _dispatch_scatter(x_hbm, p1_hbm, p2_hbm, w1_hbm, w2_hbm, xs_hbm, wp_hbm,
                      i1x, i1y, i2x, i2y, rx, ry, w1x, w1y, w2x, w2y,
                      sr0, sr1, ss0, ss1):
    wid = lax.axis_index("s") * _NC + lax.axis_index("c")
    base = wid * _TPW
    i1_v = (i1x, i1y)
    i2_v = (i2x, i2y)
    rows_v = (rx, ry)
    w1_v = (w1x, w1y)
    w2_v = (w2x, w2y)
    sr = (sr0, sr1)
    ss = (ss0, ss1)
    rh = [None] * _NDC
    sh = [None] * _NDC
    half = _H // 2

    def scatters(g):
        b = g & 1
        return [
            pltpu.async_copy(rows_v[b], xs_hbm.at[i1_v[b]], ss[b]),
            pltpu.async_copy(rows_v[b], xs_hbm.at[i2_v[b]], ss[b]),
            pltpu.async_copy(w1_v[b], wp_hbm.at[i1_v[b]], ss[b]),
            pltpu.async_copy(w2_v[b], wp_hbm.at[i2_v[b]], ss[b]),
        ]

    for g in range(_NDC):
        b = g & 1
        off = base + g * _DC
        if g >= 2:
            for h in sh[g - 2]:
                h.wait()
        pltpu.sync_copy(p1_hbm.at[pl.ds(off, _DC)], i1_v[b])
        pltpu.sync_copy(p2_hbm.at[pl.ds(off, _DC)], i2_v[b])
        pltpu.sync_copy(w1_hbm.at[pl.ds(off, _DC)], w1_v[b])
        pltpu.sync_copy(w2_hbm.at[pl.ds(off, _DC)], w2_v[b])
        rh[g] = pltpu.async_copy(x_hbm.at[pl.ds(off, _DC)], rows_v[b], sr[b])
        if g >= 1:
            rh[g - 1].wait()
            sh[g - 1] = scatters(g - 1)
    rh[_NDC - 1].wait()
    sh[_NDC - 1] = scatters(_NDC - 1)
    for h in sh[_NDC - 2]:
        h.wait()
    for h in sh[_NDC - 1]:
        h.wait()


def _gemm_body(te_ref, valid_ref, xg_ref, w_ref, b_ref, wp_ref, out_ref):
    i = pl.program_id(0)

    @pl.when(valid_ref[i] != 0)
    def _():
        acc = jnp.dot(xg_ref[...], w_ref[0], preferred_element_type=_f32)
        y = (acc + b_ref[0]) * wp_ref[...][:, 0:1]
        u = lax.bitcast_convert_type(y, _i32)
        def rne(v):
            return lax.shift_right_logical(
                v + jnp.int32(0x7FFF)
                + jnp.bitwise_and(lax.shift_right_logical(v, 16), jnp.int32(1)),
                16)
        out_ref[...] = jnp.bitwise_or(
            rne(u[:, :_H // 2]), lax.shift_left(rne(u[:, _H // 2:]), 16))


def _gemm(tile_eid, tile_valid, xs, expert_W, expert_b, wp):
    grid_spec = pltpu.PrefetchScalarGridSpec(
        num_scalar_prefetch=2,
        grid=(_NT,),
        in_specs=[
            pl.BlockSpec((_TM, _H), lambda i, te, va: (i, 0)),
            pl.BlockSpec((1, _H, _H), lambda i, te, va: (te[i], 0, 0)),
            pl.BlockSpec((1, 1, _H), lambda i, te, va: (te[i], 0, 0)),
            pl.BlockSpec((_TM, 128), lambda i, te, va: (i, 0)),
        ],
        out_specs=pl.BlockSpec((_TM, _H // 2), lambda i, te, va: (i, 0)),
    )
    return pl.pallas_call(
        _gemm_body,
        grid_spec=grid_spec,
        out_shape=jax.ShapeDtypeStruct((_P, _H // 2), _i32),
    )(tile_eid, tile_valid, xs, expert_W, expert_b.reshape(_E, 1, _H), wp)


@functools.partial(
    pl.kernel,
    mesh=_sc_mesh,
    out_type=jax.ShapeDtypeStruct((_T, _H), _f32),
    scratch_types=[
        pltpu.VMEM((_CT,), _i32), pltpu.VMEM((_CT,), _i32),
        pltpu.VMEM((_CT,), _i32), pltpu.VMEM((_CT,), _i32),
        pltpu.VMEM((_CT, _H // 2), _i32), pltpu.VMEM((_CT, _H // 2), _i32),
        pltpu.VMEM((_CT, _H // 2), _i32), pltpu.VMEM((_CT, _H // 2), _i32),
        pltpu.VMEM((_CT, _H), _f32), pltpu.VMEM((_CT, _H), _f32),
        pltpu.SemaphoreType.DMA, pltpu.SemaphoreType.DMA,
        pltpu.SemaphoreType.DMA, pltpu.SemaphoreType.DMA,
    ],
)
def _combine(yp_hbm, p1_hbm, p2_hbm, out_hbm,
             i1x, i1y, i2x, i2y, ax, ay, bx, by, ox, oy, sg0, sg1, so0, so1):
    wid = lax.axis_index("s") * _NC + lax.axis_index("c")
    base = wid * _TPW
    i1_v = (i1x, i1y)
    i2_v = (i2x, i2y)
    a_v = (ax, ay)
    b_v = (bx, by)
    o_v = (ox, oy)
    sg = (sg0, sg1)
    so = (so0, so1)
    nch = _TPW // _CT

    def compute(b):
        m16 = jnp.int32(-65536)

        def unlo(v):
            return lax.bitcast_convert_type(lax.shift_left(v, 16), _f32)

        def unhi(v):
            return lax.bitcast_convert_type(jnp.bitwise_and(v, m16), _f32)

        def trow(t, carry):
            for c in range(0, _H // 2, 16):
                sl = pl.ds(c, 16)
                va = a_v[b][t, sl]
                vb = b_v[b][t, sl]
                o_v[b][t, sl] = unlo(va) + unlo(vb)
                o_v[b][t, pl.ds(_H // 2 + c, 16)] = unhi(va) + unhi(vb)
            return carry

        lax.fori_loop(0, _CT, trow, 0)

    gha = [None] * nch
    ghb = [None] * nch
    wh = [None] * nch
    for g in range(nch):
        b = g & 1
        off = base + g * _CT
        if g >= 2:
            wh[g - 2].wait()
        pltpu.sync_copy(p1_hbm.at[pl.ds(off, _CT)], i1_v[b])
        pltpu.sync_copy(p2_hbm.at[pl.ds(off, _CT)], i2_v[b])
        gha[g] = pltpu.async_copy(yp_hbm.at[i1_v[b]], a_v[b], sg[b])
        ghb[g] = pltpu.async_copy(yp_hbm.at[i2_v[b]], b_v[b], sg[b])
        if g >= 1:
            gha[g - 1].wait()
            ghb[g - 1].wait()
            compute(1 - b)
            wh[g - 1] = pltpu.async_copy(
                o_v[1 - b], out_hbm.at[pl.ds(off - _CT, _CT)], so[1 - b])
    bl = (nch - 1) & 1
    gha[nch - 1].wait()
    ghb[nch - 1].wait()
    compute(bl)
    wh[nch - 1] = pltpu.async_copy(
        o_v[bl], out_hbm.at[pl.ds(base + (nch - 1) * _CT, _CT)], so[bl])
    wh[nch - 2].wait()
    wh[nch - 1].wait()


def kernel(x, router_W, router_b, expert_W, expert_b):
    x2 = x.reshape(_T, _H)
    e1o, e2o, w1o, w2o = _router(x2, router_W, router_b)
    e1 = e1o.reshape(_T)
    e2 = e2o.reshape(_T)

    ea = jnp.concatenate([e1, e2])
    oh = (ea[:, None] == jnp.arange(_E, dtype=_i32)[None, :]).astype(_i32)
    ranks = jnp.cumsum(oh, axis=0)
    counts = ranks[-1]
    padded = ((counts + _TM - 1) // _TM) * _TM
    pend = jnp.cumsum(padded)
    poff = pend - padded
    rank_a = jnp.take_along_axis(ranks, ea[:, None], axis=1)[:, 0]
    pos = poff[ea] + rank_a - 1
    tile_start = jnp.arange(_NT, dtype=_i32) * _TM
    tile_e = jnp.sum((tile_start[:, None] >= pend[None, :]).astype(_i32), axis=1)
    tile_eid = jnp.minimum(tile_e, _E - 1).astype(_i32)
    tile_valid = (tile_start < pend[-1]).astype(_i32)
    pos1 = pos[:_T]
    pos2 = pos[_T:]
    w1w = jnp.broadcast_to(w1o, (_T, 128))
    w2w = jnp.broadcast_to(w2o, (_T, 128))

    xs, wp = _dispatch_scatter(x2, pos1, pos2, w1w, w2w)
    yp = _gemm(tile_eid, tile_valid, xs, expert_W, expert_b, wp)
    out2 = _combine(yp, pos1, pos2)
    return out2.reshape(_B, _S, _H)


# speedup vs baseline: 1.1879x; 1.0615x over previous
import functools

import jax
import jax.numpy as jnp
from jax import lax
from jax.experimental import pallas as pl
from jax.experimental.pallas import tpu as pltpu
from jax.experimental.pallas import tpu_sc as plsc

_E = 8
_H = 1024
_K = 2
_B, _S = 4, 2048
_T = _B * _S
_A = _T * _K
_TM = 256
_P = _A + _E * _TM
_NT = _P // _TM
_RT = 1024

_NC, _NS = 2, 16
_NW = _NC * _NS
_TPW = _T // _NW
_DC = 32
_NDC = _TPW // _DC
_CT = 16

_f32 = jnp.float32
_i32 = jnp.int32


def _router_body(x_ref, w_ref, b_ref, e1_ref, e2_ref, w1_ref, w2_ref):
    s = jnp.dot(x_ref[...], w_ref[...], preferred_element_type=_f32,
                precision=lax.Precision.DEFAULT) + b_ref[...]
    m1 = jnp.max(s, axis=1, keepdims=True)
    den = jnp.sum(jnp.exp(s - m1), axis=1, keepdims=True)
    p = jnp.exp(s - m1) / den
    iot = lax.broadcasted_iota(_i32, s.shape, 1)
    p1 = jnp.max(p, axis=1, keepdims=True)
    id1 = jnp.min(jnp.where(p == p1, iot, _E), axis=1, keepdims=True)
    pm = jnp.where(iot == id1, -1.0, p)
    p2 = jnp.max(pm, axis=1, keepdims=True)
    id2 = jnp.min(jnp.where(pm == p2, iot, _E), axis=1, keepdims=True)
    tot = p1 + p2 + 1e-9
    e1_ref[...] = id1
    e2_ref[...] = id2
    w1_ref[...] = p1 / tot
    w2_ref[...] = p2 / tot


def _router(x2, router_W, router_b):
    return pl.pallas_call(
        _router_body,
        grid=(_T // _RT,),
        in_specs=[
            pl.BlockSpec((_RT, _H), lambda i: (i, 0)),
            pl.BlockSpec((_H, _E), lambda i: (0, 0)),
            pl.BlockSpec((1, _E), lambda i: (0, 0)),
        ],
        out_specs=[
            pl.BlockSpec((_RT, 1), lambda i: (i, 0)),
            pl.BlockSpec((_RT, 1), lambda i: (i, 0)),
            pl.BlockSpec((_RT, 1), lambda i: (i, 0)),
            pl.BlockSpec((_RT, 1), lambda i: (i, 0)),
        ],
        out_shape=[
            jax.ShapeDtypeStruct((_T, 1), _i32),
            jax.ShapeDtypeStruct((_T, 1), _i32),
            jax.ShapeDtypeStruct((_T, 1), _f32),
            jax.ShapeDtypeStruct((_T, 1), _f32),
        ],
    )(x2, router_W, router_b.reshape(1, _E))


_sc_mesh = plsc.VectorSubcoreMesh(core_axis_name="c", subcore_axis_name="s")


@functools.partial(
    pl.kernel,
    mesh=_sc_mesh,
    out_type=[
        jax.ShapeDtypeStruct((_P, _H), _f32),
        jax.ShapeDtypeStruct((_P, 128), _f32),
    ],
    scratch_types=[
        pltpu.VMEM((_DC,), _i32), pltpu.VMEM((_DC,), _i32),
        pltpu.VMEM((_DC,), _i32), pltpu.VMEM((_DC,), _i32),
        pltpu.VMEM((_DC, _H), _f32), pltpu.VMEM((_DC, _H), _f32),
        pltpu.VMEM((_DC, 128), _f32), pltpu.VMEM((_DC, 128), _f32),
        pltpu.VMEM((_DC, 128), _f32), pltpu.VMEM((_DC, 128), _f32),
        pltpu.SemaphoreType.DMA, pltpu.SemaphoreType.DMA,
        pltpu.SemaphoreType.DMA, pltpu.SemaphoreType.DMA,
    ],
)
def _dispatch_scatter(x_hbm, p1_hbm, p2_hbm, w1_hbm, w2_hbm, xs_hbm, wp_hbm,
                      i1x, i1y, i2x, i2y, rx, ry, w1x, w1y, w2x, w2y,
                      sr0, sr1, ss0, ss1):
    wid = lax.axis_index("s") * _NC + lax.axis_index("c")
    base = wid * _TPW
    i1_v = (i1x, i1y)
    i2_v = (i2x, i2y)
    rows_v = (rx, ry)
    w1_v = (w1x, w1y)
    w2_v = (w2x, w2y)
    sr = (sr0, sr1)
    ss = (ss0, ss1)
    rh = [None] * _NDC
    sh = [None] * _NDC
    half = _H // 2

    def scatters(g):
        b = g & 1
        return [
            pltpu.async_copy(rows_v[b], xs_hbm.at[i1_v[b]], ss[b]),
            pltpu.async_copy(rows_v[b], xs_hbm.at[i2_v[b]], ss[b]),
            pltpu.async_copy(w1_v[b], wp_hbm.at[i1_v[b]], ss[b]),
            pltpu.async_copy(w2_v[b], wp_hbm.at[i2_v[b]], ss[b]),
        ]

    for g in range(_NDC):
        b = g & 1
        off = base + g * _DC
        if g >= 2:
            for h in sh[g - 2]:
                h.wait()
        pltpu.sync_copy(p1_hbm.at[pl.ds(off, _DC)], i1_v[b])
        pltpu.sync_copy(p2_hbm.at[pl.ds(off, _DC)], i2_v[b])
        pltpu.sync_copy(w1_hbm.at[pl.ds(off, _DC)], w1_v[b])
        pltpu.sync_copy(w2_hbm.at[pl.ds(off, _DC)], w2_v[b])
        rh[g] = pltpu.async_copy(x_hbm.at[pl.ds(off, _DC)], rows_v[b], sr[b])
        if g >= 1:
            rh[g - 1].wait()
            sh[g - 1] = scatters(g - 1)
    rh[_NDC - 1].wait()
    sh[_NDC - 1] = scatters(_NDC - 1)
    for h in sh[_NDC - 2]:
        h.wait()
    for h in sh[_NDC - 1]:
        h.wait()


def _gemm_body(te_ref, valid_ref, xg_ref, w_ref, b_ref, wp_ref, out_ref):
    i = pl.program_id(0)

    @pl.when(valid_ref[i] != 0)
    def _():
        acc = jnp.dot(xg_ref[...], w_ref[0], preferred_element_type=_f32)
        out_ref[...] = (acc + b_ref[0]) * wp_ref[...][:, 0:1]


def _gemm(tile_eid, tile_valid, xs, expert_W, expert_b, wp):
    grid_spec = pltpu.PrefetchScalarGridSpec(
        num_scalar_prefetch=2,
        grid=(_NT,),
        in_specs=[
            pl.BlockSpec((_TM, _H), lambda i, te, va: (i, 0)),
            pl.BlockSpec((1, _H, _H), lambda i, te, va: (te[i], 0, 0)),
            pl.BlockSpec((1, 1, _H), lambda i, te, va: (te[i], 0, 0)),
            pl.BlockSpec((_TM, 128), lambda i, te, va: (i, 0)),
        ],
        out_specs=pl.BlockSpec((_TM, _H), lambda i, te, va: (i, 0)),
    )
    return pl.pallas_call(
        _gemm_body,
        grid_spec=grid_spec,
        out_shape=jax.ShapeDtypeStruct((_P, _H), _f32),
    )(tile_eid, tile_valid, xs, expert_W, expert_b.reshape(_E, 1, _H), wp)


@functools.partial(
    pl.kernel,
    mesh=_sc_mesh,
    out_type=jax.ShapeDtypeStruct((_T, _H), _f32),
    scratch_types=[
        pltpu.VMEM((_CT,), _i32), pltpu.VMEM((_CT,), _i32),
        pltpu.VMEM((_CT,), _i32), pltpu.VMEM((_CT,), _i32),
        pltpu.VMEM((_CT, _H), _f32), pltpu.VMEM((_CT, _H), _f32),
        pltpu.VMEM((_CT, _H), _f32), pltpu.VMEM((_CT, _H), _f32),
        pltpu.SemaphoreType.DMA, pltpu.SemaphoreType.DMA,
        pltpu.SemaphoreType.DMA, pltpu.SemaphoreType.DMA,
    ],
)
def _combine(yp_hbm, p1_hbm, p2_hbm, out_hbm,
             i1x, i1y, i2x, i2y, ax, ay, bx, by, sg0, sg1, so0, so1):
    wid = lax.axis_index("s") * _NC + lax.axis_index("c")
    base = wid * _TPW
    i1_v = (i1x, i1y)
    i2_v = (i2x, i2y)
    a_v = (ax, ay)
    b_v = (bx, by)
    sg = (sg0, sg1)
    so = (so0, so1)
    nch = _TPW // _CT

    def compute(b):
        def trow(t, carry):
            for c in range(0, _H, 16):
                sl = pl.ds(c, 16)
                a_v[b][t, sl] = a_v[b][t, sl] + b_v[b][t, sl]
            return carry

        lax.fori_loop(0, _CT, trow, 0)

    gha = [None] * nch
    ghb = [None] * nch
    wh = [None] * nch
    for g in range(nch):
        b = g & 1
        off = base + g * _CT
        if g >= 2:
            wh[g - 2].wait()
        pltpu.sync_copy(p1_hbm.at[pl.ds(off, _CT)], i1_v[b])
        pltpu.sync_copy(p2_hbm.at[pl.ds(off, _CT)], i2_v[b])
        gha[g] = pltpu.async_copy(yp_hbm.at[i1_v[b]], a_v[b], sg[b])
        ghb[g] = pltpu.async_copy(yp_hbm.at[i2_v[b]], b_v[b], sg[b])
        if g >= 1:
            gha[g - 1].wait()
            ghb[g - 1].wait()
            compute(1 - b)
            wh[g - 1] = pltpu.async_copy(
                a_v[1 - b], out_hbm.at[pl.ds(off - _CT, _CT)], so[1 - b])
    bl = (nch - 1) & 1
    gha[nch - 1].wait()
    ghb[nch - 1].wait()
    compute(bl)
    wh[nch - 1] = pltpu.async_copy(
        a_v[bl], out_hbm.at[pl.ds(base + (nch - 1) * _CT, _CT)], so[bl])
    wh[nch - 2].wait()
    wh[nch - 1].wait()


def kernel(x, router_W, router_b, expert_W, expert_b):
    x2 = x.reshape(_T, _H)
    e1o, e2o, w1o, w2o = _router(x2, router_W, router_b)
    e1 = e1o.reshape(_T)
    e2 = e2o.reshape(_T)

    ea = jnp.concatenate([e1, e2])
    oh = (ea[:, None] == jnp.arange(_E, dtype=_i32)[None, :]).astype(_i32)
    ranks = jnp.cumsum(oh, axis=0)
    counts = ranks[-1]
    padded = ((counts + _TM - 1) // _TM) * _TM
    pend = jnp.cumsum(padded)
    poff = pend - padded
    rank_a = jnp.take_along_axis(ranks, ea[:, None], axis=1)[:, 0]
    pos = poff[ea] + rank_a - 1
    tile_start = jnp.arange(_NT, dtype=_i32) * _TM
    tile_e = jnp.sum((tile_start[:, None] >= pend[None, :]).astype(_i32), axis=1)
    tile_eid = jnp.minimum(tile_e, _E - 1).astype(_i32)
    tile_valid = (tile_start < pend[-1]).astype(_i32)
    pos1 = pos[:_T]
    pos2 = pos[_T:]
    w1w = jnp.broadcast_to(w1o, (_T, 128))
    w2w = jnp.broadcast_to(w2o, (_T, 128))

    xs, wp = _dispatch_scatter(x2, pos1, pos2, w1w, w2w)
    yp = _gemm(tile_eid, tile_valid, xs, expert_W, expert_b, wp)
    out2 = _combine(yp, pos1, pos2)
    return out2.reshape(_B, _S, _H)
